# baseline (speedup 1.0000x reference)
import functools

import jax
import jax.numpy as jnp
from jax import lax
from jax.experimental import pallas as pl
from jax.experimental.pallas import tpu as pltpu
from jax.experimental.pallas import tpu_sc as plsc

NC = 2
NS = 16
NW = NC * NS
LANES = 16
EBLK = 256


def _mm_body(x_ref, w_ref, o_ref, *, relu):
    x = x_ref[...]
    if relu:
        x = jnp.maximum(x, 0.0)
    o_ref[...] = lax.dot_general(
        x, w_ref[...], (((1,), (1,)), ((), ())),
        preferred_element_type=jnp.float32)


def _matmul(x, w, relu):
    m, k = x.shape
    o = w.shape[0]
    bm = 512
    return pl.pallas_call(
        functools.partial(_mm_body, relu=relu),
        grid=(pl.cdiv(m, bm),),
        in_specs=[
            pl.BlockSpec((bm, k), lambda i: (i, 0)),
            pl.BlockSpec((o, k), lambda i: (0, 0)),
        ],
        out_specs=pl.BlockSpec((bm, o), lambda i: (i, 0)),
        out_shape=jax.ShapeDtypeStruct((m, o), jnp.float32),
    )(x, w)


def _spmm_sc(b_mat, col, vals, row, bounds, n_nodes, rows_per, pad_base):
    d = b_mat.shape[1]
    nj = d // LANES
    ng = EBLK // LANES
    npad = NW * rows_per
    mesh = plsc.VectorSubcoreMesh(
        core_axis_name="c", subcore_axis_name="s",
        num_cores=NC, num_subcores=NS)

    def body(b_hbm, col_hbm, vals_hbm, row_hbm, bounds_hbm, out_hbm,
             bounds_v, colv0, colv1, colv2, colv3, rowv0, rowv1, rowv2,
             rowv3, valv0, valv1, valv2, valv3, rowsv0, rowsv1, outv,
             gsem0, gsem1, isem0, isem1, isem2, isem3):
        colvs = (colv0, colv1, colv2, colv3)
        rowvs = (rowv0, rowv1, rowv2, rowv3)
        valvs = (valv0, valv1, valv2, valv3)
        rowsvs = (rowsv0, rowsv1)
        gsems = (gsem0, gsem1)
        isems = (isem0, isem1, isem2, isem3)

        cid = lax.axis_index("c")
        sid = lax.axis_index("s")
        wid = sid * NC + cid
        off = pl.multiple_of(wid * LANES, 8)
        pltpu.sync_copy(bounds_hbm.at[pl.ds(off, LANES)], bounds_v)
        bvec = bounds_v[pl.ds(0, LANES)]
        e_lo = bvec[0]
        e_hi = bvec[1]
        r_lo = wid * rows_per
        e0 = e_lo - lax.rem(e_lo, 8)
        nblk = lax.div(e_hi - e0 + (EBLK - 1), EBLK)

        def eb_of(b):
            return pl.multiple_of(
                jnp.where(b < nblk, e0 + b * EBLK, pad_base), 8)

        def issue_idx(b, k):
            eb = eb_of(b)
            pltpu.async_copy(col_hbm.at[pl.ds(eb, EBLK)], colvs[k], isems[k])
            pltpu.async_copy(row_hbm.at[pl.ds(eb, EBLK)], rowvs[k], isems[k])
            pltpu.async_copy(vals_hbm.at[pl.ds(eb, EBLK)], valvs[k], isems[k])

        def wait_idx(k):
            pltpu.make_async_copy(
                col_hbm.at[pl.ds(0, EBLK)], colvs[k], isems[k]).wait()
            pltpu.make_async_copy(
                row_hbm.at[pl.ds(0, EBLK)], rowvs[k], isems[k]).wait()
            pltpu.make_async_copy(
                vals_hbm.at[pl.ds(0, EBLK)], valvs[k], isems[k]).wait()

        def issue_g(k2, k4):
            pltpu.async_copy(b_hbm.at[colvs[k4]], rowsvs[k2], gsems[k2])

        def wait_g(k2):
            pltpu.make_async_copy(
                b_hbm.at[colvs[0]], rowsvs[k2], gsems[k2]).wait()

        zeros16 = jnp.zeros((LANES,), jnp.float32)

        def zrow(i, c):
            outv[pl.ds(i * LANES, LANES)] = zeros16
            return c

        lax.fori_loop(0, (rows_per + 1) * nj, zrow, 0)

        trash = r_lo + rows_per

        def compute(b, k2, k4, carry):
            eb = eb_of(b)
            glo = jnp.clip(lax.div(e_lo - eb, LANES), 0, ng)
            ghi = jnp.clip(lax.div(e_hi - eb + (LANES - 1), LANES), 0, ng)

            def grp_body(g, gcarry):
                rv = rowvs[k4][pl.ds(g * LANES, LANES)]
                vv = valvs[k4][pl.ds(g * LANES, LANES)]
                for lane in range(LANES):
                    prev = gcarry[0]
                    acc = gcarry[1:]
                    e = g * LANES + lane
                    ge = eb + e
                    r = rv[lane]
                    v = vv[lane]
                    inb = jnp.logical_and(ge >= e_lo, ge < e_hi)
                    flush = jnp.logical_and(inb, r != prev)
                    scale = jnp.where(inb, v, 0.0)
                    base = jnp.where(inb, r - r_lo, rows_per) * d
                    newacc = []
                    for j in range(nj):
                        a = (jnp.where(flush, 0.0, acc[j])
                             + scale * rowsvs[k2][e, pl.ds(j * LANES, LANES)])
                        outv[pl.ds(base + j * LANES, LANES)] = a
                        newacc.append(a)
                    gcarry = (jnp.where(inb, r, prev),) + tuple(newacc)
                return gcarry

            return lax.fori_loop(glo, ghi, grp_body, carry)

        issue_idx(0, 0)
        issue_idx(1, 1)
        wait_idx(0)
        issue_g(0, 0)

        def outer(p, carry):
            for k in range(4):
                b = p * 4 + k
                wait_idx((k + 1) % 4)
                issue_g((k + 1) % 2, (k + 1) % 4)
                issue_idx(b + 2, (k + 2) % 4)
                wait_g(k % 2)
                carry = compute(b, k % 2, k % 4, carry)
            return carry

        init = (trash,) + tuple(jnp.zeros((LANES,), jnp.float32)
                                for _ in range(nj))
        nout = lax.div(nblk + 3, 4)
        lax.fori_loop(0, nout, outer, init)
        wait_g(0)
        wait_idx(1)
        pltpu.sync_copy(outv.at[pl.ds(0, rows_per * d)],
                        out_hbm.at[pl.ds(r_lo * d, rows_per * d)])

    k = pl.kernel(
        body,
        out_type=jax.ShapeDtypeStruct((npad * d,), jnp.float32),
        mesh=mesh,
        scratch_types=[
            pltpu.VMEM((LANES,), jnp.int32),
            pltpu.VMEM((EBLK,), jnp.int32),
            pltpu.VMEM((EBLK,), jnp.int32),
            pltpu.VMEM((EBLK,), jnp.int32),
            pltpu.VMEM((EBLK,), jnp.int32),
            pltpu.VMEM((EBLK,), jnp.int32),
            pltpu.VMEM((EBLK,), jnp.int32),
            pltpu.VMEM((EBLK,), jnp.int32),
            pltpu.VMEM((EBLK,), jnp.int32),
            pltpu.VMEM((EBLK,), jnp.float32),
            pltpu.VMEM((EBLK,), jnp.float32),
            pltpu.VMEM((EBLK,), jnp.float32),
            pltpu.VMEM((EBLK,), jnp.float32),
            pltpu.VMEM((EBLK, d), jnp.float32),
            pltpu.VMEM((EBLK, d), jnp.float32),
            pltpu.VMEM(((rows_per + 1) * d,), jnp.float32),
            pltpu.SemaphoreType.DMA,
            pltpu.SemaphoreType.DMA,
            pltpu.SemaphoreType.DMA,
            pltpu.SemaphoreType.DMA,
            pltpu.SemaphoreType.DMA,
            pltpu.SemaphoreType.DMA,
        ],
    )
    return k(b_mat, col, vals, row, bounds).reshape(npad, d)


def kernel(X, W1, W2, vals, row, col):
    n, _ = X.shape
    e = row.shape[0]
    rows_per = -(-n // (NW * 8)) * 8

    r_bounds = jnp.minimum(jnp.arange(NW + 1, dtype=jnp.int32) * rows_per, n)
    bnd = jnp.searchsorted(row, r_bounds, side="left").astype(jnp.int32)
    bounds = jnp.zeros((NW, 16), jnp.int32)
    bounds = bounds.at[:, 0].set(bnd[:NW]).at[:, 1].set(bnd[1:]).reshape(-1)
    pad_base = -(-e // 8) * 8
    pad = pad_base - e + EBLK
    colp = jnp.concatenate([col, jnp.zeros((pad,), col.dtype)])
    rowp = jnp.concatenate([row, jnp.zeros((pad,), row.dtype)])
    valsp = jnp.concatenate([vals, jnp.zeros((pad,), vals.dtype)])

    h = _matmul(X, W1, relu=False)
    h = _spmm_sc(h, colp, valsp, rowp, bounds, n, rows_per, pad_base)[:n]
    h = _matmul(h, W2, relu=True)
    out = _spmm_sc(h, colp, valsp, rowp, bounds, n, rows_per, pad_base)[:n]
    return out


# speedup vs baseline: 1.3914x; 1.3914x over previous
import functools

import jax
import jax.numpy as jnp
from jax import lax
from jax.experimental import pallas as pl
from jax.experimental.pallas import tpu as pltpu
from jax.experimental.pallas import tpu_sc as plsc

NC = 2
NS = 16
NW = NC * NS
LANES = 16
EBLK = 128


def _mm_body(x_ref, w_ref, o_ref, *, relu):
    x = x_ref[...]
    if relu:
        x = jnp.maximum(x, 0.0)
    o_ref[...] = lax.dot_general(
        x, w_ref[...], (((1,), (1,)), ((), ())),
        preferred_element_type=jnp.float32)


def _matmul(x, w, relu):
    m, k = x.shape
    o = w.shape[0]
    bm = 512
    return pl.pallas_call(
        functools.partial(_mm_body, relu=relu),
        grid=(pl.cdiv(m, bm),),
        in_specs=[
            pl.BlockSpec((bm, k), lambda i: (i, 0)),
            pl.BlockSpec((o, k), lambda i: (0, 0)),
        ],
        out_specs=pl.BlockSpec((bm, o), lambda i: (i, 0)),
        out_shape=jax.ShapeDtypeStruct((m, o), jnp.float32),
    )(x, w)


def _spmm_sc(b_mat, col, vals, row, bounds, n_nodes, rows_per, pad_base):
    d = b_mat.shape[1]
    nj = d // LANES
    ng = EBLK // LANES
    npad = NW * rows_per
    mesh = plsc.VectorSubcoreMesh(
        core_axis_name="c", subcore_axis_name="s",
        num_cores=NC, num_subcores=NS)

    def body(b_hbm, col_hbm, vals_hbm, row_hbm, bounds_hbm, out_hbm,
             bounds_v, colv0, colv1, colv2, colv3, rowv0, rowv1, rowv2,
             rowv3, valv0, valv1, valv2, valv3, rowsv0, rowsv1, outv,
             gsem0, gsem1, isem0, isem1, isem2, isem3):
        colvs = (colv0, colv1, colv2, colv3)
        rowvs = (rowv0, rowv1, rowv2, rowv3)
        valvs = (valv0, valv1, valv2, valv3)
        rowsvs = (rowsv0, rowsv1)
        gsems = (gsem0, gsem1)
        isems = (isem0, isem1, isem2, isem3)

        cid = lax.axis_index("c")
        sid = lax.axis_index("s")
        wid = sid * NC + cid
        off = pl.multiple_of(wid * LANES, 8)
        pltpu.sync_copy(bounds_hbm.at[pl.ds(off, LANES)], bounds_v)
        bvec = bounds_v[pl.ds(0, LANES)]
        e_lo = bvec[0]
        e_hi = bvec[1]
        r_lo = wid * rows_per
        e0 = e_lo - lax.rem(e_lo, 8)
        nblk = lax.div(e_hi - e0 + (EBLK - 1), EBLK)

        def eb_of(b):
            return pl.multiple_of(
                jnp.where(b < nblk, e0 + b * EBLK, pad_base), 8)

        def issue_idx(b, k):
            eb = eb_of(b)
            pltpu.async_copy(col_hbm.at[pl.ds(eb, EBLK)], colvs[k], isems[k])
            pltpu.async_copy(row_hbm.at[pl.ds(eb, EBLK)], rowvs[k], isems[k])
            pltpu.async_copy(vals_hbm.at[pl.ds(eb, EBLK)], valvs[k], isems[k])

        def wait_idx(k):
            pltpu.make_async_copy(
                col_hbm.at[pl.ds(0, EBLK)], colvs[k], isems[k]).wait()
            pltpu.make_async_copy(
                row_hbm.at[pl.ds(0, EBLK)], rowvs[k], isems[k]).wait()
            pltpu.make_async_copy(
                vals_hbm.at[pl.ds(0, EBLK)], valvs[k], isems[k]).wait()

        def issue_g(k2, k4):
            pltpu.async_copy(b_hbm.at[colvs[k4]], rowsvs[k2], gsems[k2])

        def wait_g(k2):
            pltpu.make_async_copy(
                b_hbm.at[colvs[0]], rowsvs[k2], gsems[k2]).wait()

        zeros16 = jnp.zeros((LANES,), jnp.float32)

        def zrow(i, c):
            outv[pl.ds(i * LANES, LANES)] = zeros16
            return c

        lax.fori_loop(0, (rows_per + 1) * nj, zrow, 0)

        trash = r_lo + rows_per

        def compute(b, k2, k4, carry):
            eb = eb_of(b)
            glo = jnp.clip(lax.div(e_lo - eb, LANES), 0, ng)
            ghi = jnp.clip(lax.div(e_hi - eb + (LANES - 1), LANES), 0, ng)

            def grp_body(g, gcarry):
                rv = rowvs[k4][pl.ds(g * LANES, LANES)]
                vv = valvs[k4][pl.ds(g * LANES, LANES)]
                for lane in range(LANES):
                    prev = gcarry[0]
                    acc = gcarry[1:]
                    e = g * LANES + lane
                    ge = eb + e
                    r = rv[lane]
                    v = vv[lane]
                    inb = jnp.logical_and(ge >= e_lo, ge < e_hi)
                    flush = jnp.logical_and(inb, r != prev)
                    scale = jnp.where(inb, v, 0.0)
                    base = jnp.where(inb, r - r_lo, rows_per) * d
                    newacc = []
                    for j in range(nj):
                        a = (jnp.where(flush, 0.0, acc[j])
                             + scale * rowsvs[k2][e, pl.ds(j * LANES, LANES)])
                        outv[pl.ds(base + j * LANES, LANES)] = a
                        newacc.append(a)
                    gcarry = (jnp.where(inb, r, prev),) + tuple(newacc)
                return gcarry

            return lax.fori_loop(glo, ghi, grp_body, carry)

        issue_idx(0, 0)
        issue_idx(1, 1)
        wait_idx(0)
        issue_g(0, 0)

        def outer(p, carry):
            for k in range(4):
                b = p * 4 + k
                wait_idx((k + 1) % 4)
                issue_g((k + 1) % 2, (k + 1) % 4)
                issue_idx(b + 2, (k + 2) % 4)
                wait_g(k % 2)
                carry = compute(b, k % 2, k % 4, carry)
            return carry

        init = (trash,) + tuple(jnp.zeros((LANES,), jnp.float32)
                                for _ in range(nj))
        nout = lax.div(nblk + 3, 4)
        lax.fori_loop(0, nout, outer, init)
        wait_g(0)
        wait_idx(1)
        pltpu.sync_copy(outv.at[pl.ds(0, rows_per * d)],
                        out_hbm.at[pl.ds(r_lo * d, rows_per * d)])

    k = pl.kernel(
        body,
        out_type=jax.ShapeDtypeStruct((npad * d,), jnp.float32),
        mesh=mesh,
        scratch_types=[
            pltpu.VMEM((LANES,), jnp.int32),
            pltpu.VMEM((EBLK,), jnp.int32),
            pltpu.VMEM((EBLK,), jnp.int32),
            pltpu.VMEM((EBLK,), jnp.int32),
            pltpu.VMEM((EBLK,), jnp.int32),
            pltpu.VMEM((EBLK,), jnp.int32),
            pltpu.VMEM((EBLK,), jnp.int32),
            pltpu.VMEM((EBLK,), jnp.int32),
            pltpu.VMEM((EBLK,), jnp.int32),
            pltpu.VMEM((EBLK,), jnp.float32),
            pltpu.VMEM((EBLK,), jnp.float32),
            pltpu.VMEM((EBLK,), jnp.float32),
            pltpu.VMEM((EBLK,), jnp.float32),
            pltpu.VMEM((EBLK, d), jnp.float32),
            pltpu.VMEM((EBLK, d), jnp.float32),
            pltpu.VMEM(((rows_per + 1) * d,), jnp.float32),
            pltpu.SemaphoreType.DMA,
            pltpu.SemaphoreType.DMA,
            pltpu.SemaphoreType.DMA,
            pltpu.SemaphoreType.DMA,
            pltpu.SemaphoreType.DMA,
            pltpu.SemaphoreType.DMA,
        ],
    )
    return k(b_mat, col, vals, row, bounds).reshape(npad, d)


def kernel(X, W1, W2, vals, row, col):
    n, _ = X.shape
    e = row.shape[0]
    rows_per = -(-n // (NW * 8)) * 8

    r_bounds = jnp.minimum(jnp.arange(NW + 1, dtype=jnp.int32) * rows_per, n)
    bnd = jnp.searchsorted(row, r_bounds, side="left").astype(jnp.int32)
    bounds = jnp.zeros((NW, 16), jnp.int32)
    bounds = bounds.at[:, 0].set(bnd[:NW]).at[:, 1].set(bnd[1:]).reshape(-1)
    pad_base = -(-e // 8) * 8
    pad = pad_base - e + EBLK
    colp = jnp.concatenate([col, jnp.zeros((pad,), col.dtype)])
    rowp = jnp.concatenate([row, jnp.zeros((pad,), row.dtype)])
    valsp = jnp.concatenate([vals, jnp.zeros((pad,), vals.dtype)])

    h = _matmul(X, W1, relu=False)
    h = _spmm_sc(h, colp, valsp, rowp, bounds, n, rows_per, pad_base)[:n]
    h = _matmul(h, W2, relu=True)
    out = _spmm_sc(h, colp, valsp, rowp, bounds, n, rows_per, pad_base)[:n]
    return out


# speedup vs baseline: 2.2644x; 1.6274x over previous
import functools

import jax
import jax.numpy as jnp
from jax import lax
from jax.experimental import pallas as pl
from jax.experimental.pallas import tpu as pltpu
from jax.experimental.pallas import tpu_sc as plsc

NC = 2
NS = 16
NW = NC * NS
LANES = 16
EBLK = 128


def _mm_body(x_ref, w_ref, o_ref, *, relu):
    x = x_ref[...]
    if relu:
        x = jnp.maximum(x, 0.0)
    o_ref[...] = lax.dot_general(
        x, w_ref[...], (((1,), (1,)), ((), ())),
        preferred_element_type=jnp.float32)


def _matmul(x, w, relu):
    m, k = x.shape
    o = w.shape[0]
    bm = 512
    return pl.pallas_call(
        functools.partial(_mm_body, relu=relu),
        grid=(pl.cdiv(m, bm),),
        in_specs=[
            pl.BlockSpec((bm, k), lambda i: (i, 0)),
            pl.BlockSpec((o, k), lambda i: (0, 0)),
        ],
        out_specs=pl.BlockSpec((bm, o), lambda i: (i, 0)),
        out_shape=jax.ShapeDtypeStruct((m, o), jnp.float32),
    )(x, w)


def _spmm_sc(b_mat, col, vals, row, bounds, n_nodes, rows_per, pad_base):
    d = b_mat.shape[1]
    nj = d // LANES
    ng = EBLK // LANES
    npad = NW * rows_per
    mesh = plsc.VectorSubcoreMesh(
        core_axis_name="c", subcore_axis_name="s",
        num_cores=NC, num_subcores=NS)

    def body(b_hbm, col_hbm, vals_hbm, row_hbm, bounds_hbm, out_hbm,
             bounds_v, colv0, colv1, colv2, colv3, rowv0, rowv1, rowv2,
             rowv3, valv0, valv1, valv2, valv3, rowsv0, rowsv1, outv,
             gsem0, gsem1, isem0, isem1, isem2, isem3):
        colvs = (colv0, colv1, colv2, colv3)
        rowvs = (rowv0, rowv1, rowv2, rowv3)
        valvs = (valv0, valv1, valv2, valv3)
        rowsvs = (rowsv0, rowsv1)
        gsems = (gsem0, gsem1)
        isems = (isem0, isem1, isem2, isem3)

        cid = lax.axis_index("c")
        sid = lax.axis_index("s")
        wid = sid * NC + cid
        off = pl.multiple_of(wid * LANES, 8)
        pltpu.sync_copy(bounds_hbm.at[pl.ds(off, LANES)], bounds_v)
        bvec = bounds_v[pl.ds(0, LANES)]
        e_lo = bvec[0]
        e_hi = bvec[1]
        r_lo = wid * rows_per
        e0 = e_lo - lax.rem(e_lo, 8)
        nblk = lax.div(e_hi - e0 + (EBLK - 1), EBLK)

        def eb_of(b):
            return pl.multiple_of(
                jnp.where(b < nblk, e0 + b * EBLK, pad_base), 8)

        def issue_idx(b, k):
            eb = eb_of(b)
            pltpu.async_copy(col_hbm.at[pl.ds(eb, EBLK)], colvs[k], isems[k])
            pltpu.async_copy(row_hbm.at[pl.ds(eb, EBLK)], rowvs[k], isems[k])
            pltpu.async_copy(vals_hbm.at[pl.ds(eb, EBLK)], valvs[k], isems[k])

        def wait_idx(k):
            pltpu.make_async_copy(
                col_hbm.at[pl.ds(0, EBLK)], colvs[k], isems[k]).wait()
            pltpu.make_async_copy(
                row_hbm.at[pl.ds(0, EBLK)], rowvs[k], isems[k]).wait()
            pltpu.make_async_copy(
                vals_hbm.at[pl.ds(0, EBLK)], valvs[k], isems[k]).wait()

        def issue_g(k2, k4):
            pltpu.async_copy(b_hbm.at[colvs[k4]], rowsvs[k2], gsems[k2])

        def wait_g(k2):
            pltpu.make_async_copy(
                b_hbm.at[colvs[0]], rowsvs[k2], gsems[k2]).wait()

        zeros16 = jnp.zeros((LANES,), jnp.float32)

        def zrow(i, c):
            outv[pl.ds(i * LANES, LANES)] = zeros16
            return c

        lax.fori_loop(0, (rows_per + 1) * nj, zrow, 0)

        trash = r_lo + rows_per

        def compute(b, k2, k4, carry):
            eb = eb_of(b)
            glo = jnp.clip(lax.div(e_lo - eb, LANES), 0, ng)
            ghi = jnp.clip(lax.div(e_hi - eb + (LANES - 1), LANES), 0, ng)

            def grp_body(g, gcarry):
                rv = rowvs[k4][pl.ds(g * LANES, LANES)]
                vv = valvs[k4][pl.ds(g * LANES, LANES)]
                for lane in range(LANES):
                    prev = gcarry[0]
                    acc = gcarry[1:]
                    e = g * LANES + lane
                    ge = eb + e
                    r = rv[lane]
                    v = vv[lane]
                    inb = jnp.logical_and(ge >= e_lo, ge < e_hi)
                    flush = jnp.logical_and(inb, r != prev)

                    @pl.when(flush)
                    def _(prev=prev, acc=acc):
                        base = (prev - r_lo) * d
                        for j in range(nj):
                            outv[pl.ds(base + j * LANES, LANES)] = acc[j]

                    scale = jnp.where(inb, v, 0.0)
                    newacc = tuple(
                        jnp.where(flush, 0.0, acc[j])
                        + scale * rowsvs[k2][e, pl.ds(j * LANES, LANES)]
                        for j in range(nj))
                    gcarry = (jnp.where(inb, r, prev),) + tuple(newacc)
                return gcarry

            return lax.fori_loop(glo, ghi, grp_body, carry)

        issue_idx(0, 0)
        issue_idx(1, 1)
        wait_idx(0)
        issue_g(0, 0)

        def outer(p, carry):
            for k in range(4):
                b = p * 4 + k
                wait_idx((k + 1) % 4)
                issue_g((k + 1) % 2, (k + 1) % 4)
                issue_idx(b + 2, (k + 2) % 4)
                wait_g(k % 2)
                carry = compute(b, k % 2, k % 4, carry)
            return carry

        init = (trash,) + tuple(jnp.zeros((LANES,), jnp.float32)
                                for _ in range(nj))
        nout = lax.div(nblk + 3, 4)
        final = lax.fori_loop(0, nout, outer, init)
        fbase = (final[0] - r_lo) * d
        for j in range(nj):
            outv[pl.ds(fbase + j * LANES, LANES)] = final[1 + j]
        wait_g(0)
        wait_idx(1)
        pltpu.sync_copy(outv.at[pl.ds(0, rows_per * d)],
                        out_hbm.at[pl.ds(r_lo * d, rows_per * d)])

    k = pl.kernel(
        body,
        out_type=jax.ShapeDtypeStruct((npad * d,), jnp.float32),
        mesh=mesh,
        scratch_types=[
            pltpu.VMEM((LANES,), jnp.int32),
            pltpu.VMEM((EBLK,), jnp.int32),
            pltpu.VMEM((EBLK,), jnp.int32),
            pltpu.VMEM((EBLK,), jnp.int32),
            pltpu.VMEM((EBLK,), jnp.int32),
            pltpu.VMEM((EBLK,), jnp.int32),
            pltpu.VMEM((EBLK,), jnp.int32),
            pltpu.VMEM((EBLK,), jnp.int32),
            pltpu.VMEM((EBLK,), jnp.int32),
            pltpu.VMEM((EBLK,), jnp.float32),
            pltpu.VMEM((EBLK,), jnp.float32),
            pltpu.VMEM((EBLK,), jnp.float32),
            pltpu.VMEM((EBLK,), jnp.float32),
            pltpu.VMEM((EBLK, d), jnp.float32),
            pltpu.VMEM((EBLK, d), jnp.float32),
            pltpu.VMEM(((rows_per + 1) * d,), jnp.float32),
            pltpu.SemaphoreType.DMA,
            pltpu.SemaphoreType.DMA,
            pltpu.SemaphoreType.DMA,
            pltpu.SemaphoreType.DMA,
            pltpu.SemaphoreType.DMA,
            pltpu.SemaphoreType.DMA,
        ],
    )
    return k(b_mat, col, vals, row, bounds).reshape(npad, d)


def kernel(X, W1, W2, vals, row, col):
    n, _ = X.shape
    e = row.shape[0]
    rows_per = -(-n // (NW * 8)) * 8

    r_bounds = jnp.minimum(jnp.arange(NW + 1, dtype=jnp.int32) * rows_per, n)
    bnd = jnp.searchsorted(row, r_bounds, side="left").astype(jnp.int32)
    bounds = jnp.zeros((NW, 16), jnp.int32)
    bounds = bounds.at[:, 0].set(bnd[:NW]).at[:, 1].set(bnd[1:]).reshape(-1)
    pad_base = -(-e // 8) * 8
    pad = pad_base - e + EBLK
    colp = jnp.concatenate([col, jnp.zeros((pad,), col.dtype)])
    rowp = jnp.concatenate([row, jnp.zeros((pad,), row.dtype)])
    valsp = jnp.concatenate([vals, jnp.zeros((pad,), vals.dtype)])

    h = _matmul(X, W1, relu=False)
    h = _spmm_sc(h, colp, valsp, rowp, bounds, n, rows_per, pad_base)[:n]
    h = _matmul(h, W2, relu=True)
    out = _spmm_sc(h, colp, valsp, rowp, bounds, n, rows_per, pad_base)[:n]
    return out


# speedup vs baseline: 2.4557x; 1.0845x over previous
import functools

import jax
import jax.numpy as jnp
from jax import lax
from jax.experimental import pallas as pl
from jax.experimental.pallas import tpu as pltpu
from jax.experimental.pallas import tpu_sc as plsc

NC = 2
NS = 16
NW = NC * NS
LANES = 16
EBLK = 128


def _mm_body(x_ref, w_ref, o_ref, *, relu):
    x = x_ref[...]
    if relu:
        x = jnp.maximum(x, 0.0)
    o_ref[...] = lax.dot_general(
        x, w_ref[...], (((1,), (1,)), ((), ())),
        preferred_element_type=jnp.float32)


def _matmul(x, w, relu):
    m, k = x.shape
    o = w.shape[0]
    bm = 512
    return pl.pallas_call(
        functools.partial(_mm_body, relu=relu),
        grid=(pl.cdiv(m, bm),),
        in_specs=[
            pl.BlockSpec((bm, k), lambda i: (i, 0)),
            pl.BlockSpec((o, k), lambda i: (0, 0)),
        ],
        out_specs=pl.BlockSpec((bm, o), lambda i: (i, 0)),
        out_shape=jax.ShapeDtypeStruct((m, o), jnp.float32),
    )(x, w)


def _spmm_sc(b_mat, col, vals, row, bounds, n_nodes, rows_per, pad_base):
    d = b_mat.shape[1]
    nj = d // LANES
    ng = EBLK // LANES
    npad = NW * rows_per
    mesh = plsc.VectorSubcoreMesh(
        core_axis_name="c", subcore_axis_name="s",
        num_cores=NC, num_subcores=NS)

    def body(b_hbm, col_hbm, vals_hbm, row_hbm, bounds_hbm, out_hbm,
             bounds_v, colv0, colv1, rowv, valv, rowsv0, rowsv1, outv,
             gsem0, gsem1):
        colvs = (colv0, colv1)
        rowsvs = (rowsv0, rowsv1)
        gsems = (gsem0, gsem1)

        cid = lax.axis_index("c")
        sid = lax.axis_index("s")
        wid = sid * NC + cid
        off = pl.multiple_of(wid * LANES, 8)
        pltpu.sync_copy(bounds_hbm.at[pl.ds(off, LANES)], bounds_v)
        bvec = bounds_v[pl.ds(0, LANES)]
        e_lo = bvec[0]
        e_hi = bvec[1]
        r_lo = wid * rows_per
        e0 = e_lo - lax.rem(e_lo, 8)
        nblk = lax.div(e_hi - e0 + (EBLK - 1), EBLK)

        def eb_of(b):
            return pl.multiple_of(
                jnp.where(b < nblk, e0 + b * EBLK, pad_base), 8)

        def issue_g(b, k2):
            pltpu.sync_copy(col_hbm.at[pl.ds(eb_of(b), EBLK)], colvs[k2])
            pltpu.async_copy(b_hbm.at[colvs[k2]], rowsvs[k2], gsems[k2])

        def wait_g(k2):
            pltpu.make_async_copy(
                b_hbm.at[colvs[0]], rowsvs[k2], gsems[k2]).wait()

        zeros16 = jnp.zeros((LANES,), jnp.float32)

        def zrow(i, c):
            outv[pl.ds(i * LANES, LANES)] = zeros16
            return c

        lax.fori_loop(0, (rows_per + 1) * nj, zrow, 0)

        trash = r_lo + rows_per

        def run_block(b, k2, carry):
            eb = eb_of(b)
            issue_g(b + 1, 1 - k2)
            pltpu.sync_copy(row_hbm.at[pl.ds(eb, EBLK)], rowv)
            pltpu.sync_copy(vals_hbm.at[pl.ds(eb, EBLK)], valv)
            wait_g(k2)

            def grp_body(g, gcarry):
                rv = rowv[pl.ds(g * LANES, LANES)]
                vv = valv[pl.ds(g * LANES, LANES)]
                for lane in range(LANES):
                    prev = gcarry[0]
                    acc = gcarry[1:]
                    e = g * LANES + lane
                    ge = eb + e
                    r = rv[lane]
                    v = vv[lane]
                    inb = jnp.logical_and(ge >= e_lo, ge < e_hi)
                    flush = jnp.logical_and(inb, r != prev)

                    @pl.when(flush)
                    def _(prev=prev, acc=acc):
                        base = (prev - r_lo) * d
                        for j in range(nj):
                            outv[pl.ds(base + j * LANES, LANES)] = acc[j]

                    scale = jnp.where(inb, v, 0.0)
                    newacc = tuple(
                        jnp.where(flush, 0.0, acc[j])
                        + scale * rowsvs[k2][e, pl.ds(j * LANES, LANES)]
                        for j in range(nj))
                    gcarry = (jnp.where(inb, r, prev),) + newacc
                return gcarry

            return lax.fori_loop(0, EBLK // LANES, grp_body, carry)

        def outer(p, carry):
            carry = run_block(2 * p, 0, carry)
            carry = run_block(2 * p + 1, 1, carry)
            return carry

        init = (trash,) + tuple(jnp.zeros((LANES,), jnp.float32)
                                for _ in range(nj))
        issue_g(0, 0)
        nout = lax.div(nblk + 1, 2)
        final = lax.fori_loop(0, nout, outer, init)
        fbase = (final[0] - r_lo) * d
        for j in range(nj):
            outv[pl.ds(fbase + j * LANES, LANES)] = final[1 + j]
        wait_g(0)
        pltpu.sync_copy(outv.at[pl.ds(0, rows_per * d)],
                        out_hbm.at[pl.ds(r_lo * d, rows_per * d)])

    k = pl.kernel(
        body,
        out_type=jax.ShapeDtypeStruct((npad * d,), jnp.float32),
        mesh=mesh,
        scratch_types=[
            pltpu.VMEM((LANES,), jnp.int32),
            pltpu.VMEM((EBLK,), jnp.int32),
            pltpu.VMEM((EBLK,), jnp.int32),
            pltpu.VMEM((EBLK,), jnp.int32),
            pltpu.VMEM((EBLK,), jnp.float32),
            pltpu.VMEM((EBLK, d), jnp.float32),
            pltpu.VMEM((EBLK, d), jnp.float32),
            pltpu.VMEM(((rows_per + 1) * d,), jnp.float32),
            pltpu.SemaphoreType.DMA,
            pltpu.SemaphoreType.DMA,
        ],
    )
    return k(b_mat, col, vals, row, bounds).reshape(npad, d)


def kernel(X, W1, W2, vals, row, col):
    n, _ = X.shape
    e = row.shape[0]
    rows_per = -(-n // (NW * 8)) * 8

    r_bounds = jnp.minimum(jnp.arange(NW + 1, dtype=jnp.int32) * rows_per, n)
    bnd = jnp.searchsorted(row, r_bounds, side="left").astype(jnp.int32)
    bounds = jnp.zeros((NW, 16), jnp.int32)
    bounds = bounds.at[:, 0].set(bnd[:NW]).at[:, 1].set(bnd[1:]).reshape(-1)
    pad_base = -(-e // 8) * 8
    pad = pad_base - e + EBLK
    colp = jnp.concatenate([col, jnp.zeros((pad,), col.dtype)])
    rowp = jnp.concatenate([row, jnp.zeros((pad,), row.dtype)])
    valsp = jnp.concatenate([vals, jnp.zeros((pad,), vals.dtype)])

    h = _matmul(X, W1, relu=False)
    h = _spmm_sc(h, colp, valsp, rowp, bounds, n, rows_per, pad_base)[:n]
    h = _matmul(h, W2, relu=True)
    out = _spmm_sc(h, colp, valsp, rowp, bounds, n, rows_per, pad_base)[:n]
    return out


# speedup vs baseline: 2.4666x; 1.0044x over previous
import functools

import jax
import jax.numpy as jnp
from jax import lax
from jax.experimental import pallas as pl
from jax.experimental.pallas import tpu as pltpu
from jax.experimental.pallas import tpu_sc as plsc

NC = 2
NS = 16
NW = NC * NS
LANES = 16
EBLK = 128


def _mm_body(x_ref, w_ref, o_ref, *, relu):
    x = x_ref[...]
    if relu:
        x = jnp.maximum(x, 0.0)
    o_ref[...] = lax.dot_general(
        x, w_ref[...], (((1,), (1,)), ((), ())),
        preferred_element_type=jnp.float32)


def _matmul(x, w, relu):
    m, k = x.shape
    o = w.shape[0]
    bm = 512
    return pl.pallas_call(
        functools.partial(_mm_body, relu=relu),
        grid=(pl.cdiv(m, bm),),
        in_specs=[
            pl.BlockSpec((bm, k), lambda i: (i, 0)),
            pl.BlockSpec((o, k), lambda i: (0, 0)),
        ],
        out_specs=pl.BlockSpec((bm, o), lambda i: (i, 0)),
        out_shape=jax.ShapeDtypeStruct((m, o), jnp.float32),
    )(x, w)


def _spmm_sc(b_mat, col, vals, row, bounds, n_nodes, rows_per, pad_base):
    d = b_mat.shape[1]
    nj = d // LANES
    ng = EBLK // LANES
    npad = NW * rows_per
    mesh = plsc.VectorSubcoreMesh(
        core_axis_name="c", subcore_axis_name="s",
        num_cores=NC, num_subcores=NS)

    def body(b_hbm, col_hbm, vals_hbm, row_hbm, bounds_hbm, out_hbm,
             bounds_v, colv0, colv1, rowv, valv, rowsv0, rowsv1, outv,
             gsem0, gsem1):
        colvs = (colv0, colv1)
        rowsvs = (rowsv0, rowsv1)
        gsems = (gsem0, gsem1)

        cid = lax.axis_index("c")
        sid = lax.axis_index("s")
        wid = sid * NC + cid
        off = pl.multiple_of(wid * LANES, 8)
        pltpu.sync_copy(bounds_hbm.at[pl.ds(off, LANES)], bounds_v)
        bvec = bounds_v[pl.ds(0, LANES)]
        e_lo = bvec[0]
        e_hi = bvec[1]
        r_lo = wid * rows_per
        e0 = e_lo - lax.rem(e_lo, 8)
        nblk = lax.div(e_hi - e0 + (EBLK - 1), EBLK)

        def eb_of(b):
            return pl.multiple_of(
                jnp.where(b < nblk, e0 + b * EBLK, pad_base), 8)

        def issue_g(b, k2):
            pltpu.sync_copy(col_hbm.at[pl.ds(eb_of(b), EBLK)], colvs[k2])
            pltpu.async_copy(b_hbm.at[colvs[k2]], rowsvs[k2], gsems[k2])

        def wait_g(k2):
            pltpu.make_async_copy(
                b_hbm.at[colvs[0]], rowsvs[k2], gsems[k2]).wait()

        zeros16 = jnp.zeros((LANES,), jnp.float32)

        def zrow(i, c):
            outv[pl.ds(i * LANES, LANES)] = zeros16
            return c

        lax.fori_loop(0, (rows_per + 1) * nj, zrow, 0)

        trash = r_lo + rows_per

        def run_block(b, k2, carry):
            eb = eb_of(b)
            pltpu.sync_copy(row_hbm.at[pl.ds(eb, EBLK)], rowv)
            pltpu.sync_copy(vals_hbm.at[pl.ds(eb, EBLK)], valv)
            issue_g(b + 1, 1 - k2)
            wait_g(k2)

            def grp_body(g, gcarry):
                rv = rowv[pl.ds(g * LANES, LANES)]
                vv = valv[pl.ds(g * LANES, LANES)]
                for lane in range(LANES):
                    prev = gcarry[0]
                    acc = gcarry[1:]
                    e = g * LANES + lane
                    ge = eb + e
                    r = rv[lane]
                    v = vv[lane]
                    inb = jnp.logical_and(ge >= e_lo, ge < e_hi)
                    flush = jnp.logical_and(inb, r != prev)

                    @pl.when(flush)
                    def _(prev=prev, acc=acc):
                        base = (prev - r_lo) * d
                        for j in range(nj):
                            outv[pl.ds(base + j * LANES, LANES)] = acc[j]

                    scale = jnp.where(inb, v, 0.0)
                    newacc = tuple(
                        jnp.where(flush, 0.0, acc[j])
                        + scale * rowsvs[k2][e, pl.ds(j * LANES, LANES)]
                        for j in range(nj))
                    gcarry = (jnp.where(inb, r, prev),) + newacc
                return gcarry

            return lax.fori_loop(0, EBLK // LANES, grp_body, carry)

        def outer(p, carry):
            carry = run_block(2 * p, 0, carry)
            carry = run_block(2 * p + 1, 1, carry)
            return carry

        init = (trash,) + tuple(jnp.zeros((LANES,), jnp.float32)
                                for _ in range(nj))
        issue_g(0, 0)
        nout = lax.div(nblk + 1, 2)
        final = lax.fori_loop(0, nout, outer, init)
        fbase = (final[0] - r_lo) * d
        for j in range(nj):
            outv[pl.ds(fbase + j * LANES, LANES)] = final[1 + j]
        wait_g(0)
        pltpu.sync_copy(outv.at[pl.ds(0, rows_per * d)],
                        out_hbm.at[pl.ds(r_lo * d, rows_per * d)])

    k = pl.kernel(
        body,
        out_type=jax.ShapeDtypeStruct((npad * d,), jnp.float32),
        mesh=mesh,
        scratch_types=[
            pltpu.VMEM((LANES,), jnp.int32),
            pltpu.VMEM((EBLK,), jnp.int32),
            pltpu.VMEM((EBLK,), jnp.int32),
            pltpu.VMEM((EBLK,), jnp.int32),
            pltpu.VMEM((EBLK,), jnp.float32),
            pltpu.VMEM((EBLK, d), jnp.float32),
            pltpu.VMEM((EBLK, d), jnp.float32),
            pltpu.VMEM(((rows_per + 1) * d,), jnp.float32),
            pltpu.SemaphoreType.DMA,
            pltpu.SemaphoreType.DMA,
        ],
    )
    return k(b_mat, col, vals, row, bounds).reshape(npad, d)


def kernel(X, W1, W2, vals, row, col):
    n, _ = X.shape
    e = row.shape[0]
    rows_per = -(-n // (NW * 8)) * 8

    r_bounds = jnp.minimum(jnp.arange(NW + 1, dtype=jnp.int32) * rows_per, n)
    bnd = jnp.searchsorted(row, r_bounds, side="left").astype(jnp.int32)
    bounds = jnp.zeros((NW, 16), jnp.int32)
    bounds = bounds.at[:, 0].set(bnd[:NW]).at[:, 1].set(bnd[1:]).reshape(-1)
    pad_base = -(-e // 8) * 8
    pad = pad_base - e + EBLK
    colp = jnp.concatenate([col, jnp.zeros((pad,), col.dtype)])
    rowp = jnp.concatenate([row, jnp.zeros((pad,), row.dtype)])
    valsp = jnp.concatenate([vals, jnp.zeros((pad,), vals.dtype)])

    h = _matmul(X, W1, relu=False)
    h = _spmm_sc(h, colp, valsp, rowp, bounds, n, rows_per, pad_base)[:n]
    h = _matmul(h, W2, relu=True)
    out = _spmm_sc(h, colp, valsp, rowp, bounds, n, rows_per, pad_base)[:n]
    return out


# speedup vs baseline: 2.8155x; 1.1415x over previous
import functools

import jax
import jax.numpy as jnp
from jax import lax
from jax.experimental import pallas as pl
from jax.experimental.pallas import tpu as pltpu
from jax.experimental.pallas import tpu_sc as plsc

NC = 2
NS = 16
NW = NC * NS
LANES = 16
EBLK = 128


def _mm_body(x_ref, w_ref, o_ref, *, relu):
    x = x_ref[...]
    if relu:
        x = jnp.maximum(x, 0.0)
    o_ref[...] = lax.dot_general(
        x, w_ref[...], (((1,), (1,)), ((), ())),
        preferred_element_type=jnp.float32)


def _matmul(x, w, relu):
    m, k = x.shape
    o = w.shape[0]
    bm = 512
    return pl.pallas_call(
        functools.partial(_mm_body, relu=relu),
        grid=(pl.cdiv(m, bm),),
        in_specs=[
            pl.BlockSpec((bm, k), lambda i: (i, 0)),
            pl.BlockSpec((o, k), lambda i: (0, 0)),
        ],
        out_specs=pl.BlockSpec((bm, o), lambda i: (i, 0)),
        out_shape=jax.ShapeDtypeStruct((m, o), jnp.float32),
    )(x, w)


def _spmm_sc(b_mat, col, vals, row, bounds, n_nodes, rows_per, pad_base):
    d = b_mat.shape[1]
    nj = d // LANES
    ng = EBLK // LANES
    npad = NW * rows_per
    mesh = plsc.VectorSubcoreMesh(
        core_axis_name="c", subcore_axis_name="s",
        num_cores=NC, num_subcores=NS)

    def body(b_hbm, col_hbm, vals_hbm, row_hbm, bounds_hbm, out_hbm,
             bounds_v, colv, rowv, valv, rowsv, outv, gsem):
        cid = lax.axis_index("c")
        sid = lax.axis_index("s")
        wid = sid * NC + cid
        off = pl.multiple_of(wid * LANES, 8)
        pltpu.sync_copy(bounds_hbm.at[pl.ds(off, LANES)], bounds_v)
        bvec = bounds_v[pl.ds(0, LANES)]
        e_lo = bvec[0]
        e_hi = bvec[1]
        r_lo = wid * rows_per
        e0 = e_lo - lax.rem(e_lo, 8)
        nblk = lax.div(e_hi - e0 + (EBLK - 1), EBLK)

        def eb_of(b):
            return pl.multiple_of(
                jnp.where(b < nblk, e0 + b * EBLK, pad_base), 8)

        def issue_g(b, k2):
            pltpu.sync_copy(col_hbm.at[pl.ds(eb_of(b), EBLK)], colv.at[k2])
            pltpu.async_copy(b_hbm.at[colv.at[k2]], rowsv.at[k2],
                             gsem.at[k2])

        def wait_g(k2):
            pltpu.make_async_copy(
                b_hbm.at[colv.at[k2]], rowsv.at[k2], gsem.at[k2]).wait()

        zeros16 = jnp.zeros((LANES,), jnp.float32)

        def zrow(i, c):
            outv[pl.ds(i * LANES, LANES)] = zeros16
            return c

        lax.fori_loop(0, (rows_per + 1) * nj, zrow, 0)

        trash = r_lo + rows_per

        def run_block(b, carry):
            k2 = jnp.bitwise_and(b, 1)
            eb = eb_of(b)
            pltpu.sync_copy(row_hbm.at[pl.ds(eb, EBLK)], rowv)
            pltpu.sync_copy(vals_hbm.at[pl.ds(eb, EBLK)], valv)
            issue_g(b + 1, 1 - k2)
            wait_g(k2)

            def grp_body(g, gcarry):
                rv = rowv[pl.ds(g * LANES, LANES)]
                vv = valv[pl.ds(g * LANES, LANES)]
                for lane in range(LANES):
                    prev = gcarry[0]
                    acc = gcarry[1:]
                    e = g * LANES + lane
                    ge = eb + e
                    r = rv[lane]
                    v = vv[lane]
                    inb = jnp.logical_and(ge >= e_lo, ge < e_hi)
                    flush = jnp.logical_and(inb, r != prev)

                    @pl.when(flush)
                    def _(prev=prev, acc=acc):
                        base = (prev - r_lo) * d
                        for j in range(nj):
                            outv[pl.ds(base + j * LANES, LANES)] = acc[j]

                    scale = jnp.where(inb, v, 0.0)
                    newacc = tuple(
                        jnp.where(flush, 0.0, acc[j])
                        + scale * rowsv[k2, e, pl.ds(j * LANES, LANES)]
                        for j in range(nj))
                    gcarry = (jnp.where(inb, r, prev),) + newacc
                return gcarry

            return lax.fori_loop(0, EBLK // LANES, grp_body, carry)

        init = (trash,) + tuple(jnp.zeros((LANES,), jnp.float32)
                                for _ in range(nj))
        issue_g(0, 0)
        final = lax.fori_loop(0, nblk, run_block, init)
        fbase = (final[0] - r_lo) * d
        for j in range(nj):
            outv[pl.ds(fbase + j * LANES, LANES)] = final[1 + j]
        wait_g(jnp.bitwise_and(nblk, 1))
        pltpu.sync_copy(outv.at[pl.ds(0, rows_per * d)],
                        out_hbm.at[pl.ds(r_lo * d, rows_per * d)])

    k = pl.kernel(
        body,
        out_type=jax.ShapeDtypeStruct((npad * d,), jnp.float32),
        mesh=mesh,
        scratch_types=[
            pltpu.VMEM((LANES,), jnp.int32),
            pltpu.VMEM((2, EBLK), jnp.int32),
            pltpu.VMEM((EBLK,), jnp.int32),
            pltpu.VMEM((EBLK,), jnp.float32),
            pltpu.VMEM((2, EBLK, d), jnp.float32),
            pltpu.VMEM(((rows_per + 1) * d,), jnp.float32),
            pltpu.SemaphoreType.DMA((2,)),
        ],
    )
    return k(b_mat, col, vals, row, bounds).reshape(npad, d)


def kernel(X, W1, W2, vals, row, col):
    n, _ = X.shape
    e = row.shape[0]
    rows_per = -(-n // (NW * 8)) * 8

    r_bounds = jnp.minimum(jnp.arange(NW + 1, dtype=jnp.int32) * rows_per, n)
    bnd = jnp.searchsorted(row, r_bounds, side="left").astype(jnp.int32)
    bounds = jnp.zeros((NW, 16), jnp.int32)
    bounds = bounds.at[:, 0].set(bnd[:NW]).at[:, 1].set(bnd[1:]).reshape(-1)
    pad_base = -(-e // 8) * 8
    pad = pad_base - e + EBLK
    colp = jnp.concatenate([col, jnp.zeros((pad,), col.dtype)])
    rowp = jnp.concatenate([row, jnp.zeros((pad,), row.dtype)])
    valsp = jnp.concatenate([vals, jnp.zeros((pad,), vals.dtype)])

    h = _matmul(X, W1, relu=False)
    h = _spmm_sc(h, colp, valsp, rowp, bounds, n, rows_per, pad_base)[:n]
    h = _matmul(h, W2, relu=True)
    out = _spmm_sc(h, colp, valsp, rowp, bounds, n, rows_per, pad_base)[:n]
    return out


# speedup vs baseline: 3.0960x; 1.0996x over previous
import functools

import jax
import jax.numpy as jnp
from jax import lax
from jax.experimental import pallas as pl
from jax.experimental.pallas import tpu as pltpu
from jax.experimental.pallas import tpu_sc as plsc

NC = 2
NS = 16
NW = NC * NS
LANES = 16
EBLK = 128


def _mm_body(x_ref, w_ref, o_ref, *, relu):
    x = x_ref[...]
    if relu:
        x = jnp.maximum(x, 0.0)
    o_ref[...] = lax.dot_general(
        x, w_ref[...], (((1,), (1,)), ((), ())),
        preferred_element_type=jnp.float32)


def _matmul(x, w, relu):
    m, k = x.shape
    o = w.shape[0]
    bm = 512
    return pl.pallas_call(
        functools.partial(_mm_body, relu=relu),
        grid=(pl.cdiv(m, bm),),
        in_specs=[
            pl.BlockSpec((bm, k), lambda i: (i, 0)),
            pl.BlockSpec((o, k), lambda i: (0, 0)),
        ],
        out_specs=pl.BlockSpec((bm, o), lambda i: (i, 0)),
        out_shape=jax.ShapeDtypeStruct((m, o), jnp.float32),
    )(x, w)


def _spmm_sc(b_mat, col, vals, row, bounds, n_nodes, rows_per):
    d = b_mat.shape[1]
    nj = d // LANES
    npad = NW * rows_per
    mesh = plsc.VectorSubcoreMesh(
        core_axis_name="c", subcore_axis_name="s",
        num_cores=NC, num_subcores=NS)

    def body(b_hbm, col_hbm, vals_hbm, row_hbm, bounds_hbm, out_hbm,
             bounds_v, colv, rowv, valv, rowsv, outv, sem):
        cid = lax.axis_index("c")
        sid = lax.axis_index("s")
        wid = sid * NC + cid
        off = pl.multiple_of(wid * LANES, 8)
        pltpu.sync_copy(bounds_hbm.at[pl.ds(off, LANES)], bounds_v)
        bvec = bounds_v[pl.ds(0, LANES)]
        e_lo = bvec[0]
        e_hi = bvec[1]
        r_lo = wid * rows_per
        e0 = e_lo - lax.rem(e_lo, 8)
        nblk = lax.div(e_hi - e0 + (EBLK - 1), EBLK)
        rp_u = jnp.uint32(rows_per)

        zeros16 = jnp.zeros((LANES,), jnp.float32)

        def zrow(i, c):
            outv[pl.ds(i * LANES, LANES)] = zeros16
            return c

        lax.fori_loop(0, rows_per * nj, zrow, 0)

        trash = r_lo + rows_per

        def clamped_base(prev):
            lu = prev - r_lo
            return jnp.where(lu.astype(jnp.uint32) < rp_u,
                             lu, rows_per) * d

        def blk_body(b, carry):
            eb = pl.multiple_of(e0 + b * EBLK, 8)
            pltpu.sync_copy(col_hbm.at[pl.ds(eb, EBLK)], colv)
            pltpu.sync_copy(row_hbm.at[pl.ds(eb, EBLK)], rowv)
            pltpu.sync_copy(vals_hbm.at[pl.ds(eb, EBLK)], valv)
            pltpu.async_copy(b_hbm.at[colv], rowsv, sem).wait()

            def grp_body(g, gcarry):
                rv = rowv[pl.ds(g * LANES, LANES)]
                vv = valv[pl.ds(g * LANES, LANES)]
                for lane in range(LANES):
                    prev = gcarry[0]
                    acc = gcarry[1:]
                    e = g * LANES + lane
                    r = rv[lane]
                    v = vv[lane]
                    flush = r != prev

                    @pl.when(flush)
                    def _(prev=prev, acc=acc):
                        base = clamped_base(prev)
                        for j in range(nj):
                            outv[pl.ds(base + j * LANES, LANES)] = acc[j]

                    newacc = tuple(
                        jnp.where(flush, 0.0, acc[j])
                        + v * rowsv[e, pl.ds(j * LANES, LANES)]
                        for j in range(nj))
                    gcarry = (r,) + newacc
                return gcarry

            return lax.fori_loop(0, EBLK // LANES, grp_body, carry)

        init = (trash,) + tuple(jnp.zeros((LANES,), jnp.float32)
                                for _ in range(nj))
        final = lax.fori_loop(0, nblk, blk_body, init)
        fbase = clamped_base(final[0])
        for j in range(nj):
            outv[pl.ds(fbase + j * LANES, LANES)] = final[1 + j]
        pltpu.sync_copy(outv.at[pl.ds(0, rows_per * d)],
                        out_hbm.at[pl.ds(r_lo * d, rows_per * d)])

    k = pl.kernel(
        body,
        out_type=jax.ShapeDtypeStruct((npad * d,), jnp.float32),
        mesh=mesh,
        scratch_types=[
            pltpu.VMEM((LANES,), jnp.int32),
            pltpu.VMEM((EBLK,), jnp.int32),
            pltpu.VMEM((EBLK,), jnp.int32),
            pltpu.VMEM((EBLK,), jnp.float32),
            pltpu.VMEM((EBLK, d), jnp.float32),
            pltpu.VMEM(((rows_per + 1) * d,), jnp.float32),
            pltpu.SemaphoreType.DMA,
        ],
    )
    return k(b_mat, col, vals, row, bounds).reshape(npad, d)


def kernel(X, W1, W2, vals, row, col):
    n, _ = X.shape
    e = row.shape[0]
    rows_per = -(-n // (NW * 8)) * 8

    r_bounds = jnp.minimum(jnp.arange(NW + 1, dtype=jnp.int32) * rows_per, n)
    bnd = jnp.searchsorted(row, r_bounds, side="left").astype(jnp.int32)
    bounds = jnp.zeros((NW, 16), jnp.int32)
    bounds = bounds.at[:, 0].set(bnd[:NW]).at[:, 1].set(bnd[1:]).reshape(-1)
    pad = EBLK + 8
    colp = jnp.concatenate([col, jnp.zeros((pad,), col.dtype)])
    rowp = jnp.concatenate([row, jnp.full((pad,), n, row.dtype)])
    valsp = jnp.concatenate([vals, jnp.zeros((pad,), vals.dtype)])

    h = _matmul(X, W1, relu=False)
    h = _spmm_sc(h, colp, valsp, rowp, bounds, n, rows_per)[:n]
    h = _matmul(h, W2, relu=True)
    out = _spmm_sc(h, colp, valsp, rowp, bounds, n, rows_per)[:n]
    return out
